# baseline (speedup 1.0000x reference)
import functools

import jax
import jax.numpy as jnp
from jax import lax
from jax.experimental import pallas as pl
from jax.experimental.pallas import tpu as pltpu
from jax.experimental.pallas import tpu_sc as plsc

BATCH = 16384
FIELDS = 26
DIM = 64
PDIM = 128
NUM_CORES = 2
NUM_SUBCORES = 16
NW = NUM_CORES * NUM_SUBCORES
TOTAL = BATCH * FIELDS
CHUNK = 128
CHUNKS = TOTAL // (NW * CHUNK)
ROWS_PER_W = CHUNKS * CHUNK
NBUF = 4
LOOKAHEAD = 2
LANES = 16


def _make_kernel():
    mesh = plsc.VectorSubcoreMesh(core_axis_name="c", subcore_axis_name="s")

    @functools.partial(
        pl.kernel,
        mesh=mesh,
        out_type=jax.ShapeDtypeStruct((TOTAL, PDIM), jnp.float32),
        scratch_types=(
            [pltpu.VMEM((ROWS_PER_W,), jnp.int32),
             pltpu.VMEM((NBUF, CHUNK), jnp.int32)]
            + [pltpu.VMEM((CHUNK, PDIM), jnp.float32) for _ in range(NBUF)]
            + [pltpu.SemaphoreType.DMA((NBUF,)), pltpu.SemaphoreType.DMA((NBUF,))]
        ),
    )
    def body(x_hbm, w_hbm, out_hbm, idx_v, q_v, *rest):
        rows = rest[:NBUF]
        gsem, wsem = rest[NBUF], rest[NBUF + 1]
        wid = lax.axis_index("s") * NUM_CORES + lax.axis_index("c")
        base = wid * ROWS_PER_W
        pltpu.sync_copy(x_hbm.at[pl.ds(base, ROWS_PER_W)], idx_v)

        def fire_gather(c, b):
            for k in range(CHUNK // LANES):
                q_v[b, pl.ds(k * LANES, LANES)] = lax.shift_right_logical(
                    idx_v[pl.ds(c * CHUNK + k * LANES, LANES)], 1)
            pltpu.async_copy(w_hbm.at[q_v.at[b]], rows[b], gsem.at[b])

        def wait_gather(b):
            pltpu.make_async_copy(
                w_hbm.at[q_v.at[0]], rows[b], gsem.at[b]).wait()

        def fixup(c, b):
            def grp_fix(i, carry):
                par16 = jnp.bitwise_and(
                    idx_v[pl.ds(c * CHUNK + i * LANES, LANES)], 1)
                for j in range(LANES):
                    @pl.when(par16[j] == 1)
                    def _():
                        row = i * LANES + j
                        for k in range(DIM // LANES):
                            rows[b][row, pl.ds(k * LANES, LANES)] = (
                                rows[b][row, pl.ds(DIM + k * LANES, LANES)])

                return carry

            lax.fori_loop(0, CHUNK // LANES, grp_fix, 0)

        def fire_write(c, b):
            pltpu.async_copy(
                rows[b], out_hbm.at[pl.ds(base + c * CHUNK, CHUNK)],
                wsem.at[b])

        def wait_write(b):
            pltpu.make_async_copy(
                rows[b], out_hbm.at[pl.ds(0, CHUNK)], wsem.at[b]).wait()

        for b in range(LOOKAHEAD):
            fire_gather(b, b)

        for b in range(NBUF):
            wait_gather(b)
            fixup(b, b)
            fire_write(b, b)
            bb = (b + LOOKAHEAD) % NBUF
            if b >= NBUF - LOOKAHEAD:
                wait_write(bb)
            fire_gather(b + LOOKAHEAD, bb)

        def block(gi, carry):
            g = gi * NBUF
            for b in range(NBUF):
                c = g + b
                wait_gather(b)
                fixup(c, b)
                fire_write(c, b)
                bb = (b + LOOKAHEAD) % NBUF
                wait_write(bb)
                fire_gather(c + LOOKAHEAD, bb)
            return carry

        lax.fori_loop(1, CHUNKS // NBUF - 1, block, 0)

        g = CHUNKS - NBUF
        for b in range(NBUF):
            c = g + b
            wait_gather(b)
            fixup(c, b)
            fire_write(c, b)
            if b < LOOKAHEAD:
                bb = (b + LOOKAHEAD) % NBUF
                wait_write(bb)
                fire_gather(c + LOOKAHEAD, bb)

        for b in range(NBUF):
            wait_write(b)

    return body


_kern = _make_kernel()


def kernel(x, weight):
    xf = x.reshape(TOTAL).astype(jnp.int32)
    wp = weight.reshape(500000, PDIM)
    out = _kern(xf, wp)
    return out[:, :DIM].reshape(BATCH, FIELDS, DIM)


# speedup vs baseline: 1.1361x; 1.1361x over previous
import functools

import jax
import jax.numpy as jnp
from jax import lax
from jax.experimental import pallas as pl
from jax.experimental.pallas import tpu as pltpu
from jax.experimental.pallas import tpu_sc as plsc

BATCH = 16384
FIELDS = 26
DIM = 64
NUM_CORES = 2
NUM_SUBCORES = 16
NW = NUM_CORES * NUM_SUBCORES
TOTAL = BATCH * FIELDS
CHUNK = 128
CHUNKS = TOTAL // (NW * CHUNK)
ROWS_PER_W = CHUNKS * CHUNK
NBUF = 8
LOOKAHEAD = 4


def _make_kernel():
    mesh = plsc.VectorSubcoreMesh(core_axis_name="c", subcore_axis_name="s")

    @functools.partial(
        pl.kernel,
        mesh=mesh,
        out_type=jax.ShapeDtypeStruct((TOTAL, DIM), jnp.float32),
        scratch_types=(
            [pltpu.VMEM((ROWS_PER_W,), jnp.int32)]
            + [pltpu.VMEM((CHUNK, DIM), jnp.float32) for _ in range(NBUF)]
            + [pltpu.SemaphoreType.DMA((NBUF,)), pltpu.SemaphoreType.DMA((NBUF,))]
        ),
        compiler_params=pltpu.CompilerParams(use_tc_tiling_on_sc=False),
    )
    def body(x_hbm, w_hbm, out_hbm, idx_v, *rest):
        rows = rest[:NBUF]
        gsem, wsem = rest[NBUF], rest[NBUF + 1]
        wid = lax.axis_index("s") * NUM_CORES + lax.axis_index("c")
        base = wid * ROWS_PER_W
        pltpu.sync_copy(x_hbm.at[pl.ds(base, ROWS_PER_W)], idx_v)

        def fire_gather(c, b):
            pltpu.async_copy(
                w_hbm.at[idx_v.at[pl.ds(c * CHUNK, CHUNK)]], rows[b],
                gsem.at[b])

        def wait_gather(b):
            pltpu.make_async_copy(
                w_hbm.at[idx_v.at[pl.ds(0, CHUNK)]], rows[b],
                gsem.at[b]).wait()

        def fire_write(c, b):
            pltpu.async_copy(
                rows[b], out_hbm.at[pl.ds(base + c * CHUNK, CHUNK)],
                wsem.at[b])

        def wait_write(b):
            pltpu.make_async_copy(
                rows[b], out_hbm.at[pl.ds(0, CHUNK)], wsem.at[b]).wait()

        for b in range(LOOKAHEAD):
            fire_gather(b, b)

        for b in range(NBUF):
            wait_gather(b)
            fire_write(b, b)
            bb = (b + LOOKAHEAD) % NBUF
            if b >= NBUF - LOOKAHEAD:
                wait_write(bb)
            fire_gather(b + LOOKAHEAD, bb)

        def block(gi, carry):
            g = gi * NBUF
            for b in range(NBUF):
                c = g + b
                wait_gather(b)
                fire_write(c, b)
                bb = (b + LOOKAHEAD) % NBUF
                wait_write(bb)
                fire_gather(c + LOOKAHEAD, bb)
            return carry

        lax.fori_loop(1, CHUNKS // NBUF - 1, block, 0)

        g = CHUNKS - NBUF
        for b in range(NBUF):
            c = g + b
            wait_gather(b)
            fire_write(c, b)
            if b < LOOKAHEAD:
                bb = (b + LOOKAHEAD) % NBUF
                wait_write(bb)
                fire_gather(c + LOOKAHEAD, bb)

        for b in range(NBUF):
            wait_write(b)

    return body


_kern = _make_kernel()


def kernel(x, weight):
    xf = x.reshape(TOTAL).astype(jnp.int32)
    out = _kern(xf, weight)
    return out.reshape(BATCH, FIELDS, DIM)
